# baseline (speedup 1.0000x reference)
import functools

import jax
import jax.numpy as jnp
from jax.experimental import pallas as pl
from jax.experimental.pallas import tpu as pltpu
from jax.experimental.pallas import tpu_sc as plsc

D_NODE = 128
LATENT = 128
D_OUT = 16


def _precompute_tables(x, w_rs):
    n = x.shape[0]
    blk = 2000
    nblk = n // blk

    def body(x_ref, w_ref, o_ref):
        o_ref[...] = jnp.dot(x_ref[...], w_ref[...],
                             preferred_element_type=jnp.float32)

    return pl.pallas_call(
        body,
        grid=(2, nblk),
        in_specs=[
            pl.BlockSpec((blk, D_NODE), lambda t, i: (i, 0)),
            pl.BlockSpec((D_NODE, LATENT), lambda t, i: (t, 0)),
        ],
        out_specs=pl.BlockSpec((blk, LATENT), lambda t, i: (t * nblk + i, 0)),
        out_shape=jax.ShapeDtypeStruct((2 * n, LATENT), jnp.float32),
    )(x, w_rs)


def _sc_gather(table, idx):
    b = idx.shape[0]
    d = table.shape[1]
    window = 256
    idx2 = idx.reshape(1, b)
    mesh = plsc.VectorSubcoreMesh(core_axis_name="core",
                                  subcore_axis_name="subcore")

    @functools.partial(
        pl.kernel,
        out_type=jax.ShapeDtypeStruct((b, d), table.dtype),
        mesh=mesh,
    )
    def k(t_hbm, i_hbm, o_hbm):
        def body(i_vmem, o_vmem):
            pltpu.sync_copy(t_hbm.at[i_vmem.at[0]], o_vmem)

        pltpu.emit_pipeline(
            body,
            grid=(b // window,),
            in_specs=[pl.BlockSpec((1, window), index_map=lambda i: (0, i))],
            out_specs=[pl.BlockSpec((window, d), index_map=lambda i: (i, 0))],
            core_axis_name=("core", "subcore"),
            dimension_semantics=(pltpu.PARALLEL,),
        )(i_hbm, o_hbm)

    return k(table, idx2)


def _mlp_tail(g, ea_t, u, w1e, w1u, b1, w2t, b2_c, gamma_c, beta_c):
    e = ea_t.shape[1]
    blk = 2560 if e % 2560 == 0 else 1280
    nblk = e // blk

    def body(gr_ref, gs_ref, ea_ref, u_ref, w1e_ref, w1u_ref, b1_ref,
             w2t_ref, b2_ref, gamma_ref, beta_ref, o_ref):
        h = gr_ref[...] + gs_ref[...]
        h += jax.lax.dot_general(
            ea_ref[...], w1e_ref[...], (((0,), (0,)), ((), ())),
            preferred_element_type=jnp.float32)
        h += jnp.dot(u_ref[...], w1u_ref[...],
                     preferred_element_type=jnp.float32)
        h += b1_ref[...]
        h = jnp.maximum(h, 0.0)
        h2 = jax.lax.dot_general(
            w2t_ref[...], h, (((1,), (1,)), ((), ())),
            preferred_element_type=jnp.float32)
        h2 += b2_ref[...]
        h2 = jnp.maximum(h2, 0.0)
        mean = jnp.mean(h2, axis=0, keepdims=True)
        c = h2 - mean
        var = jnp.mean(c * c, axis=0, keepdims=True)
        o_ref[...] = c / jnp.sqrt(var + 1e-5) * gamma_ref[...] + beta_ref[...]

    return pl.pallas_call(
        body,
        grid=(nblk,),
        in_specs=[
            pl.BlockSpec((blk, LATENT), lambda i: (i, 0)),
            pl.BlockSpec((blk, LATENT), lambda i: (nblk + i, 0)),
            pl.BlockSpec((D_OUT, blk), lambda i: (0, i)),
            pl.BlockSpec((1, D_OUT), lambda i: (0, 0)),
            pl.BlockSpec((D_OUT, LATENT), lambda i: (0, 0)),
            pl.BlockSpec((D_OUT, LATENT), lambda i: (0, 0)),
            pl.BlockSpec((1, LATENT), lambda i: (0, 0)),
            pl.BlockSpec((D_OUT, LATENT), lambda i: (0, 0)),
            pl.BlockSpec((D_OUT, 1), lambda i: (0, 0)),
            pl.BlockSpec((D_OUT, 1), lambda i: (0, 0)),
            pl.BlockSpec((D_OUT, 1), lambda i: (0, 0)),
        ],
        out_specs=pl.BlockSpec((D_OUT, blk), lambda i: (0, i)),
        out_shape=jax.ShapeDtypeStruct((D_OUT, e), jnp.float32),
    )(g, g, ea_t, u, w1e, w1u, b1, w2t, b2_c, gamma_c, beta_c)


def kernel(x, edge_index, edge_attr, u, W1, b1, W2, b2, gamma, beta):
    n = x.shape[0]
    e = edge_attr.shape[0]
    row = edge_index[0].astype(jnp.int32)
    col = edge_index[1].astype(jnp.int32)

    w_rs = W1[: 2 * D_NODE]
    w1e = W1[2 * D_NODE: 2 * D_NODE + D_OUT]
    w1u = W1[2 * D_NODE + D_OUT:]
    b1r = b1.reshape(1, LATENT)
    w2t = W2.T
    b2c = b2.reshape(D_OUT, 1)
    gammac = gamma.reshape(D_OUT, 1)
    betac = beta.reshape(D_OUT, 1)
    ea_t = edge_attr.T

    table = _precompute_tables(x, w_rs)

    nchunk = 2
    ec = e // nchunk
    outs = []
    for c in range(nchunk):
        lo = c * ec
        idx_c = jnp.concatenate(
            [jax.lax.dynamic_slice_in_dim(col, lo, ec),
             jax.lax.dynamic_slice_in_dim(row, lo, ec) + n])
        g_c = _sc_gather(table, idx_c)
        ea_c = jax.lax.dynamic_slice_in_dim(ea_t, lo, ec, axis=1)
        outs.append(_mlp_tail(g_c, ea_c, u, w1e, w1u,
                              b1r, w2t, b2c, gammac, betac))
    out_t = jnp.concatenate(outs, axis=1)
    return out_t.T


# speedup vs baseline: 1.2256x; 1.2256x over previous
import functools

import jax
import jax.numpy as jnp
from jax.experimental import pallas as pl
from jax.experimental.pallas import tpu as pltpu
from jax.experimental.pallas import tpu_sc as plsc

D_NODE = 128
LATENT = 128
D_OUT = 16


def _precompute_tables(x, w_rs):
    n = x.shape[0]
    blk = 2000
    nblk = n // blk

    def body(x_ref, w_ref, o_ref):
        o_ref[...] = jnp.dot(x_ref[...], w_ref[...],
                             preferred_element_type=jnp.float32)

    return pl.pallas_call(
        body,
        grid=(2, nblk),
        in_specs=[
            pl.BlockSpec((blk, D_NODE), lambda t, i: (i, 0)),
            pl.BlockSpec((D_NODE, LATENT), lambda t, i: (1 - t, 0)),
        ],
        out_specs=pl.BlockSpec((blk, LATENT), lambda t, i: (t * nblk + i, 0)),
        out_shape=jax.ShapeDtypeStruct((2 * n, LATENT), jnp.float32),
    )(x, w_rs)


def _sc_gather(table, idx2):
    b = idx2.shape[1]
    d = table.shape[1]
    window = 256
    mesh = plsc.VectorSubcoreMesh(core_axis_name="core",
                                  subcore_axis_name="subcore")

    @functools.partial(
        pl.kernel,
        out_type=jax.ShapeDtypeStruct((b, d), table.dtype),
        mesh=mesh,
    )
    def k(t_hbm, i_hbm, o_hbm):
        def body(i_vmem, o_vmem):
            pltpu.sync_copy(t_hbm.at[i_vmem.at[0]], o_vmem)

        pltpu.emit_pipeline(
            body,
            grid=(b // window,),
            in_specs=[pl.BlockSpec((1, window), index_map=lambda i: (0, i))],
            out_specs=[pl.BlockSpec((window, d), index_map=lambda i: (i, 0))],
            core_axis_name=("core", "subcore"),
            dimension_semantics=(pltpu.PARALLEL,),
        )(i_hbm, o_hbm)

    return k(table, idx2)


def _mlp_tail(g, ea_t, u, w1e, w1u, b1, w2t, b2_c, gamma_c, beta_c):
    e = ea_t.shape[1]
    blk = 3200
    nblk = e // blk

    def body(gr_ref, gs_ref, ea_ref, u_ref, w1e_ref, w1u_ref, b1_ref,
             w2t_ref, b2_ref, gamma_ref, beta_ref, o_ref):
        h = gr_ref[...] + gs_ref[...]
        h += jax.lax.dot_general(
            ea_ref[...], w1e_ref[...], (((0,), (0,)), ((), ())),
            preferred_element_type=jnp.float32)
        h += jnp.dot(u_ref[...], w1u_ref[...],
                     preferred_element_type=jnp.float32)
        h += b1_ref[...]
        h = jnp.maximum(h, 0.0)
        h2 = jax.lax.dot_general(
            w2t_ref[...], h, (((1,), (1,)), ((), ())),
            preferred_element_type=jnp.float32)
        h2 += b2_ref[...]
        h2 = jnp.maximum(h2, 0.0)
        mean = jnp.mean(h2, axis=0, keepdims=True)
        c = h2 - mean
        var = jnp.mean(c * c, axis=0, keepdims=True)
        o_ref[...] = c / jnp.sqrt(var + 1e-5) * gamma_ref[...] + beta_ref[...]

    return pl.pallas_call(
        body,
        grid=(nblk,),
        in_specs=[
            pl.BlockSpec((blk, LATENT), lambda i: (i, 0)),
            pl.BlockSpec((blk, LATENT), lambda i: (nblk + i, 0)),
            pl.BlockSpec((D_OUT, blk), lambda i: (0, i)),
            pl.BlockSpec((1, D_OUT), lambda i: (0, 0)),
            pl.BlockSpec((D_OUT, LATENT), lambda i: (0, 0)),
            pl.BlockSpec((D_OUT, LATENT), lambda i: (0, 0)),
            pl.BlockSpec((1, LATENT), lambda i: (0, 0)),
            pl.BlockSpec((D_OUT, LATENT), lambda i: (0, 0)),
            pl.BlockSpec((D_OUT, 1), lambda i: (0, 0)),
            pl.BlockSpec((D_OUT, 1), lambda i: (0, 0)),
            pl.BlockSpec((D_OUT, 1), lambda i: (0, 0)),
        ],
        out_specs=pl.BlockSpec((D_OUT, blk), lambda i: (0, i)),
        out_shape=jax.ShapeDtypeStruct((D_OUT, e), jnp.float32),
    )(g, g, ea_t, u, w1e, w1u, b1, w2t, b2_c, gamma_c, beta_c)


def kernel(x, edge_index, edge_attr, u, W1, b1, W2, b2, gamma, beta):
    n = x.shape[0]
    e = edge_attr.shape[0]

    adj = edge_index.astype(jnp.int32) + jnp.array([[0], [n]], jnp.int32)
    idx2 = adj.reshape(1, 2 * e)

    w_rs = W1[: 2 * D_NODE]
    w1e = W1[2 * D_NODE: 2 * D_NODE + D_OUT]
    w1u = W1[2 * D_NODE + D_OUT:]

    table = _precompute_tables(x, w_rs)
    g = _sc_gather(table, idx2)
    out_t = _mlp_tail(g, edge_attr.T, u, w1e, w1u,
                      b1.reshape(1, LATENT), W2.T, b2.reshape(D_OUT, 1),
                      gamma.reshape(D_OUT, 1), beta.reshape(D_OUT, 1))
    return out_t.T
